# baseline (speedup 1.0000x reference)
import functools

import jax
import jax.numpy as jnp
from jax import lax
from jax.experimental import pallas as pl
from jax.experimental.pallas import tpu as pltpu
from jax.experimental.pallas import tpu_sc as plsc

N = 4096
M = 1024
D = 256
EA = 65536
EH = 32768
EPS_BN = 1e-5

NC = 2
NS = 16
L = 16

ZB = 16384
EAT = EA // NS
EHT = EH // NS
HALF_ROWS = N // NC
A_WORDS = (N + 1) * N

_SC_MESH = plsc.VectorSubcoreMesh(core_axis_name="c", subcore_axis_name="s")


def _zero_fill(ref, words):
    z = jnp.zeros((L,), jnp.float32)

    def body(i, _):
        ref[pl.ds(i * 4 * L, L)] = z
        ref[pl.ds(i * 4 * L + L, L)] = z
        ref[pl.ds(i * 4 * L + 2 * L, L)] = z
        ref[pl.ds(i * 4 * L + 3 * L, L)] = z
        return 0

    lax.fori_loop(0, words // (4 * L), body, 0)


def _k1_body(x_ref, w1_ref, bias_ref, wv_ref, xn_ref, vec_ref):
    xn = lax.dot_general(x_ref[...], w1_ref[...],
                         (((1,), (1,)), ((), ())),
                         preferred_element_type=jnp.float32)
    xn = xn + bias_ref[...][0:1, :]
    xn_ref[...] = xn
    vec_ref[...] = jnp.dot(xn, wv_ref[...],
                           preferred_element_type=jnp.float32) + \
        bias_ref[...][1:2, 0:128]


def _k1(x, W1_w, W1_b, Wn_w, Wn_b, We_w, We_b):
    wv = jnp.zeros((D, 128), jnp.float32)
    wv = wv.at[:, 0].set(Wn_w[0, :D])
    wv = wv.at[:, 1].set(Wn_w[0, D:])
    wv = wv.at[:, 2].set(We_w[0, :D])
    bias = jnp.zeros((2, D), jnp.float32)
    bias = bias.at[0, :].set(W1_b)
    bias = bias.at[1, 0].set(Wn_b[0])
    bias = bias.at[1, 2].set(We_b[0])
    blk = 256
    grid = N // blk
    return pl.pallas_call(
        _k1_body,
        grid=(grid,),
        in_specs=[
            pl.BlockSpec((blk, D), lambda i: (i, 0)),
            pl.BlockSpec((D, D), lambda i: (0, 0)),
            pl.BlockSpec((2, D), lambda i: (0, 0)),
            pl.BlockSpec((D, 128), lambda i: (0, 0)),
        ],
        out_specs=[
            pl.BlockSpec((blk, D), lambda i: (i, 0)),
            pl.BlockSpec((blk, 128), lambda i: (i, 0)),
        ],
        out_shape=[
            jax.ShapeDtypeStruct((N, D), jnp.float32),
            jax.ShapeDtypeStruct((N, 128), jnp.float32),
        ],
    )(x, W1_w, bias, wv)


CH = 8
CROWS = 256
CWORDS = CROWS * N
ECHUNK = EA // NS


def _k2_body(a_hbm, b_hbm, src_hbm, dst_hbm, out_hbm,
             zbuf, a_v, b_v, src_v, dst_v, idx_v, val_v, a_sh, sem):
    c = lax.axis_index("c")
    s = lax.axis_index("s")
    _zero_fill(zbuf, ZB)
    base = s * ECHUNK
    pltpu.sync_copy(src_hbm.at[pl.ds(base, ECHUNK)], src_v)
    pltpu.sync_copy(dst_hbm.at[pl.ds(base, ECHUNK)], dst_v)
    pltpu.sync_copy(a_hbm, a_v)
    pltpu.sync_copy(b_hbm, b_v)

    def vbody(i, _):
        s16 = src_v[pl.ds(i * L, L)]
        d16 = dst_v[pl.ds(i * L, L)]
        av = plsc.load_gather(a_v, [s16])
        bv = plsc.load_gather(b_v, [d16])
        val_v[pl.ds(i * L, L)] = 1.0 / (1.0 + jnp.exp(-(av + bv)))
        return 0

    lax.fori_loop(0, ECHUNK // L, vbody, 0)
    slab = CWORDS // NS
    trash = CWORDS + s * ECHUNK
    for r in range(CH):
        lo = (c * CH + r) * CROWS
        zc = [
            pltpu.async_copy(zbuf, a_sh.at[pl.ds(s * slab + j * ZB, ZB)], sem)
            for j in range(slab // ZB)
        ]
        for cp in zc:
            cp.wait()
        plsc.subcore_barrier()

        def ibody(i, _):
            s16 = src_v[pl.ds(i * L, L)]
            d16 = dst_v[pl.ds(i * L, L)]
            idx = (s16 - lo) * N + d16
            keep = (s16 >= lo) & (s16 < lo + CROWS)
            idx_v[pl.ds(i * L, L)] = jnp.where(
                keep, idx, trash + i * L + lax.iota(jnp.int32, L))
            return 0

        lax.fori_loop(0, ECHUNK // L, ibody, 0)
        pltpu.sync_copy(val_v, a_sh.at[idx_v])
        plsc.subcore_barrier()
        pltpu.sync_copy(a_sh.at[pl.ds(s * slab, slab)],
                        out_hbm.at[pl.ds(lo * N + s * slab, slab)])


def _k2(aN, bN, src, dst):
    f = pl.kernel(
        _k2_body,
        out_type=jax.ShapeDtypeStruct((N * N,), jnp.float32),
        mesh=_SC_MESH,
        compiler_params=pltpu.CompilerParams(needs_layout_passes=False),
        scratch_types=[
            pltpu.VMEM((ZB,), jnp.float32),
            pltpu.VMEM((N,), jnp.float32),
            pltpu.VMEM((N,), jnp.float32),
            pltpu.VMEM((ECHUNK,), jnp.int32),
            pltpu.VMEM((ECHUNK,), jnp.int32),
            pltpu.VMEM((ECHUNK,), jnp.int32),
            pltpu.VMEM((ECHUNK,), jnp.float32),
            pltpu.VMEM_SHARED((CWORDS + NS * ECHUNK,), jnp.float32),
            pltpu.SemaphoreType.DMA,
        ],
    )
    return f(aN, bN, src, dst)


def _k3_body(a_ref, h_ref, out_ref):
    out_ref[...] = jnp.dot(a_ref[...], h_ref[...],
                           preferred_element_type=jnp.float32)


def _k3(a2d, H):
    blk = 256
    return pl.pallas_call(
        _k3_body,
        grid=(N // blk,),
        in_specs=[
            pl.BlockSpec((blk, N), lambda i: (i, 0)),
            pl.BlockSpec((N, M), lambda i: (0, 0)),
        ],
        out_specs=pl.BlockSpec((blk, M), lambda i: (i, 0)),
        out_shape=jax.ShapeDtypeStruct((N, M), jnp.float32),
    )(a2d, H)


QROWS_P = M // (2 * NC)
QWORDS_P = QROWS_P * N

def _k4_body(b_hbm, hv_hbm, he_hbm, pt_hbm,
             zbuf, hv_v, he_v, widx_v, w_v, e_v, s_loc, p_v, pidx_v,
             s_sh, pt_sh, sem):
    c = lax.axis_index("c")
    s = lax.axis_index("s")
    _zero_fill(zbuf, ZB)

    @pl.when(s == 0)
    def _():
        pltpu.sync_copy(zbuf.at[pl.ds(0, M)], s_sh)

    slab = QWORDS_P // NS
    pt_zero = [
        pltpu.async_copy(zbuf, pt_sh.at[pl.ds(s * slab + j * ZB, ZB)], sem)
        for j in range(slab // ZB)
    ]
    for cp in pt_zero:
        cp.wait()
    base = s * EHT
    pltpu.sync_copy(hv_hbm.at[pl.ds(base, EHT)], hv_v)
    pltpu.sync_copy(he_hbm.at[pl.ds(base, EHT)], he_v)

    def mkidx(i, _):
        hv16 = hv_v[pl.ds(i * L, L)]
        he16 = he_v[pl.ds(i * L, L)]
        widx_v[pl.ds(i * L, L)] = hv16 * M + he16
        return 0

    lax.fori_loop(0, EHT // L, mkidx, 0)
    pltpu.async_copy(b_hbm.at[widx_v], w_v, sem).wait()

    def expb(i, _):
        e_v[pl.ds(i * L, L)] = jnp.exp(w_v[pl.ds(i * L, L)])
        return 0

    lax.fori_loop(0, EHT // L, expb, 0)
    plsc.subcore_barrier()
    pltpu.sync_copy(e_v, s_sh.at[he_v], add=True)
    plsc.subcore_barrier()
    pltpu.sync_copy(s_sh, s_loc)

    def pbody(i, _):
        he16 = he_v[pl.ds(i * L, L)]
        se = plsc.load_gather(s_loc, [he16])
        p_v[pl.ds(i * L, L)] = e_v[pl.ds(i * L, L)] / (se + 1e-12)
        return 0

    lax.fori_loop(0, EHT // L, pbody, 0)

    trash = QWORDS_P + s * EHT

    for r in range(2):
        q_lo = (c * 2 + r) * QROWS_P

        def qidx(i, _):
            hv16 = hv_v[pl.ds(i * L, L)]
            he16 = he_v[pl.ds(i * L, L)]
            idx = (he16 - q_lo) * N + hv16
            keep = (he16 >= q_lo) & (he16 < q_lo + QROWS_P)
            pidx_v[pl.ds(i * L, L)] = jnp.where(
                keep, idx, trash + i * L + lax.iota(jnp.int32, L))
            return 0

        lax.fori_loop(0, EHT // L, qidx, 0)
        pltpu.sync_copy(p_v, pt_sh.at[pidx_v], add=True)
        plsc.subcore_barrier()
        pltpu.sync_copy(pt_sh.at[pl.ds(s * slab, slab)],
                        pt_hbm.at[pl.ds(q_lo * N + s * slab, slab)])
        if r == 0:
            rez = [
                pltpu.async_copy(zbuf,
                                 pt_sh.at[pl.ds(s * slab + j * ZB, ZB)], sem)
                for j in range(slab // ZB)
            ]
            for cp in rez:
                cp.wait()
            plsc.subcore_barrier()


def _k4(b_flat, Hv, He):
    f = pl.kernel(
        _k4_body,
        out_type=jax.ShapeDtypeStruct((M * N,), jnp.float32),
        mesh=_SC_MESH,
        compiler_params=pltpu.CompilerParams(needs_layout_passes=False),
        scratch_types=[
            pltpu.VMEM((ZB,), jnp.float32),
            pltpu.VMEM((EHT,), jnp.int32),
            pltpu.VMEM((EHT,), jnp.int32),
            pltpu.VMEM((EHT,), jnp.int32),
            pltpu.VMEM((EHT,), jnp.float32),
            pltpu.VMEM((EHT,), jnp.float32),
            pltpu.VMEM((M,), jnp.float32),
            pltpu.VMEM((EHT,), jnp.float32),
            pltpu.VMEM((EHT,), jnp.int32),
            pltpu.VMEM_SHARED((M,), jnp.float32),
            pltpu.VMEM_SHARED((QWORDS_P + EH,), jnp.float32),
            pltpu.SemaphoreType.DMA,
        ],
    )
    return f(b_flat, Hv, He)


def _k5_body(pt_ref, xn_ref, we2_ref, xl_ref, v_ref):
    xl = jnp.dot(pt_ref[...], xn_ref[...], preferred_element_type=jnp.float32)
    xl_ref[...] = xl
    v_ref[...] = jnp.sum(xl * we2_ref[...][0:1, :], axis=1, keepdims=True)


def _k5(pt2d, x_n, we2row):
    return pl.pallas_call(
        _k5_body,
        grid=(1,),
        in_specs=[
            pl.BlockSpec((M, N), lambda i: (0, 0)),
            pl.BlockSpec((N, D), lambda i: (0, 0)),
            pl.BlockSpec((1, D), lambda i: (0, 0)),
        ],
        out_specs=[
            pl.BlockSpec((M, D), lambda i: (0, 0)),
            pl.BlockSpec((M, 1), lambda i: (0, 0)),
        ],
        out_shape=[
            jax.ShapeDtypeStruct((M, D), jnp.float32),
            jax.ShapeDtypeStruct((M, 1), jnp.float32),
        ],
    )(pt2d, x_n, we2row)


QROWS_Q = N // (2 * NC)
QWORDS_Q = QROWS_Q * M

def _k6_body(u_hbm, v_hbm, hv_hbm, he_hbm, q_hbm,
             zbuf, u_v, v_v, hv_v, he_v, e2_v, s2_loc, q_v, qidx_v,
             s2_sh, q_sh, sem):
    c = lax.axis_index("c")
    s = lax.axis_index("s")
    _zero_fill(zbuf, ZB)

    @pl.when(s == 0)
    def _():
        pltpu.sync_copy(zbuf.at[pl.ds(0, N)], s2_sh)

    slab = QWORDS_Q // NS
    qz = [
        pltpu.async_copy(zbuf, q_sh.at[pl.ds(s * slab + j * ZB, ZB)], sem)
        for j in range(slab // ZB)
    ]
    for cp in qz:
        cp.wait()
    base = s * EHT
    pltpu.sync_copy(hv_hbm.at[pl.ds(base, EHT)], hv_v)
    pltpu.sync_copy(he_hbm.at[pl.ds(base, EHT)], he_v)
    pltpu.sync_copy(u_hbm, u_v)
    pltpu.sync_copy(v_hbm, v_v)

    def ebody(i, _):
        hv16 = hv_v[pl.ds(i * L, L)]
        he16 = he_v[pl.ds(i * L, L)]
        uv = plsc.load_gather(u_v, [hv16]) + plsc.load_gather(v_v, [he16])
        ae = 1.0 / (1.0 + jnp.exp(-uv))
        e2_v[pl.ds(i * L, L)] = jnp.exp(ae)
        return 0

    lax.fori_loop(0, EHT // L, ebody, 0)
    plsc.subcore_barrier()
    pltpu.sync_copy(e2_v, s2_sh.at[hv_v], add=True)
    plsc.subcore_barrier()
    pltpu.sync_copy(s2_sh, s2_loc)

    def qbody(i, _):
        hv16 = hv_v[pl.ds(i * L, L)]
        se = plsc.load_gather(s2_loc, [hv16])
        q_v[pl.ds(i * L, L)] = e2_v[pl.ds(i * L, L)] / (se + 1e-12)
        return 0

    lax.fori_loop(0, EHT // L, qbody, 0)

    trash = QWORDS_Q + s * EHT

    for r in range(2):
        q_lo = (c * 2 + r) * QROWS_Q

        def qidx(i, _):
            hv16 = hv_v[pl.ds(i * L, L)]
            he16 = he_v[pl.ds(i * L, L)]
            idx = (hv16 - q_lo) * M + he16
            keep = (hv16 >= q_lo) & (hv16 < q_lo + QROWS_Q)
            qidx_v[pl.ds(i * L, L)] = jnp.where(
                keep, idx, trash + i * L + lax.iota(jnp.int32, L))
            return 0

        lax.fori_loop(0, EHT // L, qidx, 0)
        pltpu.sync_copy(q_v, q_sh.at[qidx_v], add=True)
        plsc.subcore_barrier()
        pltpu.sync_copy(q_sh.at[pl.ds(s * slab, slab)],
                        q_hbm.at[pl.ds(q_lo * M + s * slab, slab)])
        if r == 0:
            rez = [
                pltpu.async_copy(zbuf,
                                 q_sh.at[pl.ds(s * slab + j * ZB, ZB)], sem)
                for j in range(slab // ZB)
            ]
            for cp in rez:
                cp.wait()
            plsc.subcore_barrier()


def _k6(uN, v, Hv, He):
    f = pl.kernel(
        _k6_body,
        out_type=jax.ShapeDtypeStruct((N * M,), jnp.float32),
        mesh=_SC_MESH,
        compiler_params=pltpu.CompilerParams(needs_layout_passes=False),
        scratch_types=[
            pltpu.VMEM((ZB,), jnp.float32),
            pltpu.VMEM((N,), jnp.float32),
            pltpu.VMEM((M,), jnp.float32),
            pltpu.VMEM((EHT,), jnp.int32),
            pltpu.VMEM((EHT,), jnp.int32),
            pltpu.VMEM((EHT,), jnp.float32),
            pltpu.VMEM((N,), jnp.float32),
            pltpu.VMEM((EHT,), jnp.float32),
            pltpu.VMEM((EHT,), jnp.int32),
            pltpu.VMEM_SHARED((N,), jnp.float32),
            pltpu.VMEM_SHARED((QWORDS_Q + EH,), jnp.float32),
            pltpu.SemaphoreType.DMA,
        ],
    )
    return f(uN, v, Hv, He)


def _k7_body(q_ref, xl_ref, x_ref, sc_ref, out_ref):
    acc = jnp.dot(q_ref[...], xl_ref[...], preferred_element_type=jnp.float32)
    t = jnp.maximum(acc + x_ref[...], 0.0)
    out_ref[...] = t * sc_ref[...][0:1, :] + sc_ref[...][1:2, :]


def _k7(q2d, xl, x, scale, shift):
    blk = 256
    sc = jnp.stack([scale, shift], axis=0)
    return pl.pallas_call(
        _k7_body,
        grid=(N // blk,),
        in_specs=[
            pl.BlockSpec((blk, M), lambda i: (i, 0)),
            pl.BlockSpec((M, D), lambda i: (0, 0)),
            pl.BlockSpec((blk, D), lambda i: (i, 0)),
            pl.BlockSpec((2, D), lambda i: (0, 0)),
        ],
        out_specs=pl.BlockSpec((blk, D), lambda i: (i, 0)),
        out_shape=jax.ShapeDtypeStruct((N, D), jnp.float32),
    )(q2d, xl, x, sc)


def kernel(x, H_edge_index, H_edge_weight, A_edge_index, A_edge_weight, H, A,
           X_L, W1_w, W1_b, W2_w, W2_b, Wn_w, Wn_b, We_w, We_b,
           bn_gamma, bn_beta, bn_mean, bn_var):
    src = A_edge_index[0]
    dst = A_edge_index[1]
    Hv = H_edge_index[0]
    He = H_edge_index[1]

    x_n, vec = _k1(x, W1_w, W1_b, Wn_w, Wn_b, We_w, We_b)
    aN = vec[:, 0]
    bN = vec[:, 1]
    uN = vec[:, 2]

    a_flat = _k2(aN, bN, src, dst)
    a2d = a_flat.reshape(N, N)
    B = _k3(a2d, H)

    pt_flat = _k4(B.reshape(N * M), Hv, He)
    pt2d = pt_flat.reshape(M, N)

    xl, v2 = _k5(pt2d, x_n, We_w[:, D:])
    v = v2[:, 0]

    q_flat = _k6(uN, v, Hv, He)
    q2d = q_flat.reshape(N, M)

    scale = bn_gamma / jnp.sqrt(bn_var + EPS_BN)
    shift = bn_beta - bn_mean * scale
    x_out = _k7(q2d, xl, x, scale, shift)
    return (x_out, xl)


# speedup vs baseline: 1.0350x; 1.0350x over previous
import functools

import jax
import jax.numpy as jnp
from jax import lax
from jax.experimental import pallas as pl
from jax.experimental.pallas import tpu as pltpu
from jax.experimental.pallas import tpu_sc as plsc

N = 4096
M = 1024
D = 256
EA = 65536
EH = 32768
EPS_BN = 1e-5

NC = 2
NS = 16
L = 16

ZB = 16384
EAT = EA // NS
EHT = EH // NS
HALF_ROWS = N // NC
A_WORDS = (N + 1) * N

_SC_MESH = plsc.VectorSubcoreMesh(core_axis_name="c", subcore_axis_name="s")


def _zero_fill(ref, words):
    z = jnp.zeros((L,), jnp.float32)

    def body(i, _):
        ref[pl.ds(i * 4 * L, L)] = z
        ref[pl.ds(i * 4 * L + L, L)] = z
        ref[pl.ds(i * 4 * L + 2 * L, L)] = z
        ref[pl.ds(i * 4 * L + 3 * L, L)] = z
        return 0

    lax.fori_loop(0, words // (4 * L), body, 0)


def _k1_body(x_ref, w1_ref, b1_ref, wv_ref, bias_ref, xn_ref, vec_ref):
    xn = lax.dot_general(x_ref[...], w1_ref[...],
                         (((1,), (1,)), ((), ())),
                         preferred_element_type=jnp.float32)
    xn = xn + b1_ref[...][0:1, :]
    xn_ref[...] = xn
    vec_ref[...] = lax.dot_general(wv_ref[...], xn,
                                   (((1,), (1,)), ((), ())),
                                   preferred_element_type=jnp.float32) + \
        bias_ref[...][:, 0:1]


def _k1(x, W1_w, W1_b, Wn_w, Wn_b, We_w, We_b):
    wv8 = jnp.zeros((8, D), jnp.float32)
    wv8 = wv8.at[0, :].set(Wn_w[0, :D])
    wv8 = wv8.at[1, :].set(Wn_w[0, D:])
    wv8 = wv8.at[2, :].set(We_w[0, :D])
    bias8 = jnp.zeros((8, 1), jnp.float32)
    bias8 = bias8.at[0, 0].set(Wn_b[0])
    bias8 = bias8.at[2, 0].set(We_b[0])
    b1 = W1_b.reshape(1, D)
    blk = 256
    grid = N // blk
    return pl.pallas_call(
        _k1_body,
        grid=(grid,),
        in_specs=[
            pl.BlockSpec((blk, D), lambda i: (i, 0)),
            pl.BlockSpec((D, D), lambda i: (0, 0)),
            pl.BlockSpec((1, D), lambda i: (0, 0)),
            pl.BlockSpec((8, D), lambda i: (0, 0)),
            pl.BlockSpec((8, 1), lambda i: (0, 0)),
        ],
        out_specs=[
            pl.BlockSpec((blk, D), lambda i: (i, 0)),
            pl.BlockSpec((8, blk), lambda i: (0, i)),
        ],
        out_shape=[
            jax.ShapeDtypeStruct((N, D), jnp.float32),
            jax.ShapeDtypeStruct((8, N), jnp.float32),
        ],
    )(x, W1_w, b1, wv8, bias8)


CH = 8
CROWS = 256
CWORDS = CROWS * N
ECHUNK = EA // NS


def _k2_body(vec_hbm, src_hbm, dst_hbm, out_hbm,
             zbuf, a_v, b_v, src_v, dst_v, idx_v, val_v, a_sh, sem):
    c = lax.axis_index("c")
    s = lax.axis_index("s")
    _zero_fill(zbuf, ZB)
    base = s * ECHUNK
    pltpu.sync_copy(src_hbm.at[pl.ds(base, ECHUNK)], src_v)
    pltpu.sync_copy(dst_hbm.at[pl.ds(base, ECHUNK)], dst_v)
    pltpu.sync_copy(vec_hbm.at[0], a_v)
    pltpu.sync_copy(vec_hbm.at[1], b_v)

    def vbody(i, _):
        s16 = src_v[pl.ds(i * L, L)]
        d16 = dst_v[pl.ds(i * L, L)]
        av = plsc.load_gather(a_v, [s16])
        bv = plsc.load_gather(b_v, [d16])
        val_v[pl.ds(i * L, L)] = 1.0 / (1.0 + jnp.exp(-(av + bv)))
        return 0

    lax.fori_loop(0, ECHUNK // L, vbody, 0)
    slab = CWORDS // NS
    trash = CWORDS + s * ECHUNK
    for r in range(CH):
        lo = (c * CH + r) * CROWS
        zc = [
            pltpu.async_copy(zbuf, a_sh.at[pl.ds(s * slab + j * ZB, ZB)], sem)
            for j in range(slab // ZB)
        ]
        for cp in zc:
            cp.wait()
        plsc.subcore_barrier()

        def ibody(i, _):
            s16 = src_v[pl.ds(i * L, L)]
            d16 = dst_v[pl.ds(i * L, L)]
            idx = (s16 - lo) * N + d16
            keep = (s16 >= lo) & (s16 < lo + CROWS)
            idx_v[pl.ds(i * L, L)] = jnp.where(
                keep, idx, trash + i * L + lax.iota(jnp.int32, L))
            return 0

        lax.fori_loop(0, ECHUNK // L, ibody, 0)
        pltpu.sync_copy(val_v, a_sh.at[idx_v])
        plsc.subcore_barrier()
        pltpu.sync_copy(a_sh.at[pl.ds(s * slab, slab)],
                        out_hbm.at[pl.ds(lo * N + s * slab, slab)])


def _k2(vec, src, dst):
    f = pl.kernel(
        _k2_body,
        out_type=jax.ShapeDtypeStruct((N * N,), jnp.float32),
        mesh=_SC_MESH,
        compiler_params=pltpu.CompilerParams(needs_layout_passes=False),
        scratch_types=[
            pltpu.VMEM((ZB,), jnp.float32),
            pltpu.VMEM((N,), jnp.float32),
            pltpu.VMEM((N,), jnp.float32),
            pltpu.VMEM((ECHUNK,), jnp.int32),
            pltpu.VMEM((ECHUNK,), jnp.int32),
            pltpu.VMEM((ECHUNK,), jnp.int32),
            pltpu.VMEM((ECHUNK,), jnp.float32),
            pltpu.VMEM_SHARED((CWORDS + NS * ECHUNK,), jnp.float32),
            pltpu.SemaphoreType.DMA,
        ],
    )
    return f(vec, src, dst)


def _k3_body(a_ref, h_ref, out_ref):
    out_ref[...] = jnp.dot(a_ref[...].astype(jnp.bfloat16), h_ref[...],
                           preferred_element_type=jnp.float32)


def _k3(a2d, H):
    blk = 256
    return pl.pallas_call(
        _k3_body,
        grid=(N // blk,),
        in_specs=[
            pl.BlockSpec((blk, N), lambda i: (i, 0)),
            pl.BlockSpec((N, M), lambda i: (0, 0)),
        ],
        out_specs=pl.BlockSpec((blk, M), lambda i: (i, 0)),
        out_shape=jax.ShapeDtypeStruct((N, M), jnp.float32),
    )(a2d, H.astype(jnp.bfloat16))


QROWS_P = M // (2 * NC)
QWORDS_P = QROWS_P * N

def _k4_body(b_hbm, hv_hbm, he_hbm, pt_hbm,
             zbuf, hv_v, he_v, widx_v, w_v, e_v, s_loc, p_v, pidx_v,
             s_sh, pt_sh, sem):
    c = lax.axis_index("c")
    s = lax.axis_index("s")
    _zero_fill(zbuf, ZB)

    @pl.when(s == 0)
    def _():
        pltpu.sync_copy(zbuf.at[pl.ds(0, M)], s_sh)

    slab = QWORDS_P // NS
    pt_zero = [
        pltpu.async_copy(zbuf, pt_sh.at[pl.ds(s * slab + j * ZB, ZB)], sem)
        for j in range(slab // ZB)
    ]
    for cp in pt_zero:
        cp.wait()
    base = s * EHT
    pltpu.sync_copy(hv_hbm.at[pl.ds(base, EHT)], hv_v)
    pltpu.sync_copy(he_hbm.at[pl.ds(base, EHT)], he_v)

    def mkidx(i, _):
        hv16 = hv_v[pl.ds(i * L, L)]
        he16 = he_v[pl.ds(i * L, L)]
        widx_v[pl.ds(i * L, L)] = hv16 * M + he16
        return 0

    lax.fori_loop(0, EHT // L, mkidx, 0)
    pltpu.async_copy(b_hbm.at[widx_v], w_v, sem).wait()

    def expb(i, _):
        e_v[pl.ds(i * L, L)] = jnp.exp(w_v[pl.ds(i * L, L)])
        return 0

    lax.fori_loop(0, EHT // L, expb, 0)
    plsc.subcore_barrier()
    pltpu.sync_copy(e_v, s_sh.at[he_v], add=True)
    plsc.subcore_barrier()
    pltpu.sync_copy(s_sh, s_loc)

    def pbody(i, _):
        he16 = he_v[pl.ds(i * L, L)]
        se = plsc.load_gather(s_loc, [he16])
        p_v[pl.ds(i * L, L)] = e_v[pl.ds(i * L, L)] / (se + 1e-12)
        return 0

    lax.fori_loop(0, EHT // L, pbody, 0)

    trash = QWORDS_P + s * EHT

    for r in range(2):
        q_lo = (c * 2 + r) * QROWS_P

        def qidx(i, _):
            hv16 = hv_v[pl.ds(i * L, L)]
            he16 = he_v[pl.ds(i * L, L)]
            idx = (he16 - q_lo) * N + hv16
            keep = (he16 >= q_lo) & (he16 < q_lo + QROWS_P)
            pidx_v[pl.ds(i * L, L)] = jnp.where(
                keep, idx, trash + i * L + lax.iota(jnp.int32, L))
            return 0

        lax.fori_loop(0, EHT // L, qidx, 0)
        pltpu.sync_copy(p_v, pt_sh.at[pidx_v], add=True)
        plsc.subcore_barrier()
        pltpu.sync_copy(pt_sh.at[pl.ds(s * slab, slab)],
                        pt_hbm.at[pl.ds(q_lo * N + s * slab, slab)])
        if r == 0:
            rez = [
                pltpu.async_copy(zbuf,
                                 pt_sh.at[pl.ds(s * slab + j * ZB, ZB)], sem)
                for j in range(slab // ZB)
            ]
            for cp in rez:
                cp.wait()
            plsc.subcore_barrier()


def _k4(b_flat, Hv, He):
    f = pl.kernel(
        _k4_body,
        out_type=jax.ShapeDtypeStruct((M * N,), jnp.float32),
        mesh=_SC_MESH,
        compiler_params=pltpu.CompilerParams(needs_layout_passes=False),
        scratch_types=[
            pltpu.VMEM((ZB,), jnp.float32),
            pltpu.VMEM((EHT,), jnp.int32),
            pltpu.VMEM((EHT,), jnp.int32),
            pltpu.VMEM((EHT,), jnp.int32),
            pltpu.VMEM((EHT,), jnp.float32),
            pltpu.VMEM((EHT,), jnp.float32),
            pltpu.VMEM((M,), jnp.float32),
            pltpu.VMEM((EHT,), jnp.float32),
            pltpu.VMEM((EHT,), jnp.int32),
            pltpu.VMEM_SHARED((M,), jnp.float32),
            pltpu.VMEM_SHARED((QWORDS_P + EH,), jnp.float32),
            pltpu.SemaphoreType.DMA,
        ],
    )
    return f(b_flat, Hv, He)


def _k5_body(pt_ref, xn_ref, we2_ref, xl_ref, v_ref):
    xl = jnp.dot(pt_ref[...], xn_ref[...], preferred_element_type=jnp.float32)
    xl_ref[...] = xl
    v_ref[...] = jnp.sum(xl * we2_ref[...][0:1, :], axis=1, keepdims=True)


def _k5(pt2d, x_n, we2row):
    return pl.pallas_call(
        _k5_body,
        grid=(1,),
        in_specs=[
            pl.BlockSpec((M, N), lambda i: (0, 0)),
            pl.BlockSpec((N, D), lambda i: (0, 0)),
            pl.BlockSpec((1, D), lambda i: (0, 0)),
        ],
        out_specs=[
            pl.BlockSpec((M, D), lambda i: (0, 0)),
            pl.BlockSpec((M, 1), lambda i: (0, 0)),
        ],
        out_shape=[
            jax.ShapeDtypeStruct((M, D), jnp.float32),
            jax.ShapeDtypeStruct((M, 1), jnp.float32),
        ],
    )(pt2d, x_n, we2row)


QROWS_Q = N // (2 * NC)
QWORDS_Q = QROWS_Q * M

def _k6_body(vec_hbm, v_hbm, hv_hbm, he_hbm, q_hbm,
             zbuf, u_v, v_v, hv_v, he_v, e2_v, s2_loc, q_v, qidx_v,
             s2_sh, q_sh, sem):
    c = lax.axis_index("c")
    s = lax.axis_index("s")
    _zero_fill(zbuf, ZB)

    @pl.when(s == 0)
    def _():
        pltpu.sync_copy(zbuf.at[pl.ds(0, N)], s2_sh)

    slab = QWORDS_Q // NS
    qz = [
        pltpu.async_copy(zbuf, q_sh.at[pl.ds(s * slab + j * ZB, ZB)], sem)
        for j in range(slab // ZB)
    ]
    for cp in qz:
        cp.wait()
    base = s * EHT
    pltpu.sync_copy(hv_hbm.at[pl.ds(base, EHT)], hv_v)
    pltpu.sync_copy(he_hbm.at[pl.ds(base, EHT)], he_v)
    pltpu.sync_copy(vec_hbm.at[2], u_v)
    pltpu.sync_copy(v_hbm, v_v)

    def ebody(i, _):
        hv16 = hv_v[pl.ds(i * L, L)]
        he16 = he_v[pl.ds(i * L, L)]
        uv = plsc.load_gather(u_v, [hv16]) + plsc.load_gather(v_v, [he16])
        ae = 1.0 / (1.0 + jnp.exp(-uv))
        e2_v[pl.ds(i * L, L)] = jnp.exp(ae)
        return 0

    lax.fori_loop(0, EHT // L, ebody, 0)
    plsc.subcore_barrier()
    pltpu.sync_copy(e2_v, s2_sh.at[hv_v], add=True)
    plsc.subcore_barrier()
    pltpu.sync_copy(s2_sh, s2_loc)

    def qbody(i, _):
        hv16 = hv_v[pl.ds(i * L, L)]
        se = plsc.load_gather(s2_loc, [hv16])
        q_v[pl.ds(i * L, L)] = e2_v[pl.ds(i * L, L)] / (se + 1e-12)
        return 0

    lax.fori_loop(0, EHT // L, qbody, 0)

    trash = QWORDS_Q + s * EHT

    for r in range(2):
        q_lo = (c * 2 + r) * QROWS_Q

        def qidx(i, _):
            hv16 = hv_v[pl.ds(i * L, L)]
            he16 = he_v[pl.ds(i * L, L)]
            idx = (hv16 - q_lo) * M + he16
            keep = (hv16 >= q_lo) & (hv16 < q_lo + QROWS_Q)
            qidx_v[pl.ds(i * L, L)] = jnp.where(
                keep, idx, trash + i * L + lax.iota(jnp.int32, L))
            return 0

        lax.fori_loop(0, EHT // L, qidx, 0)
        pltpu.sync_copy(q_v, q_sh.at[qidx_v], add=True)
        plsc.subcore_barrier()
        pltpu.sync_copy(q_sh.at[pl.ds(s * slab, slab)],
                        q_hbm.at[pl.ds(q_lo * M + s * slab, slab)])
        if r == 0:
            rez = [
                pltpu.async_copy(zbuf,
                                 q_sh.at[pl.ds(s * slab + j * ZB, ZB)], sem)
                for j in range(slab // ZB)
            ]
            for cp in rez:
                cp.wait()
            plsc.subcore_barrier()


def _k6(vec, v, Hv, He):
    f = pl.kernel(
        _k6_body,
        out_type=jax.ShapeDtypeStruct((N * M,), jnp.float32),
        mesh=_SC_MESH,
        compiler_params=pltpu.CompilerParams(needs_layout_passes=False),
        scratch_types=[
            pltpu.VMEM((ZB,), jnp.float32),
            pltpu.VMEM((N,), jnp.float32),
            pltpu.VMEM((M,), jnp.float32),
            pltpu.VMEM((EHT,), jnp.int32),
            pltpu.VMEM((EHT,), jnp.int32),
            pltpu.VMEM((EHT,), jnp.float32),
            pltpu.VMEM((N,), jnp.float32),
            pltpu.VMEM((EHT,), jnp.float32),
            pltpu.VMEM((EHT,), jnp.int32),
            pltpu.VMEM_SHARED((N,), jnp.float32),
            pltpu.VMEM_SHARED((QWORDS_Q + EH,), jnp.float32),
            pltpu.SemaphoreType.DMA,
        ],
    )
    return f(vec, v, Hv, He)


def _k7_body(q_ref, xl_ref, x_ref, sc_ref, out_ref):
    acc = jnp.dot(q_ref[...], xl_ref[...], preferred_element_type=jnp.float32)
    t = jnp.maximum(acc + x_ref[...], 0.0)
    out_ref[...] = t * sc_ref[...][0:1, :] + sc_ref[...][1:2, :]


def _k7(q2d, xl, x, scale, shift):
    blk = 256
    sc = jnp.stack([scale, shift], axis=0)
    return pl.pallas_call(
        _k7_body,
        grid=(N // blk,),
        in_specs=[
            pl.BlockSpec((blk, M), lambda i: (i, 0)),
            pl.BlockSpec((M, D), lambda i: (0, 0)),
            pl.BlockSpec((blk, D), lambda i: (i, 0)),
            pl.BlockSpec((2, D), lambda i: (0, 0)),
        ],
        out_specs=pl.BlockSpec((blk, D), lambda i: (i, 0)),
        out_shape=jax.ShapeDtypeStruct((N, D), jnp.float32),
    )(q2d, xl, x, sc)


def kernel(x, H_edge_index, H_edge_weight, A_edge_index, A_edge_weight, H, A,
           X_L, W1_w, W1_b, W2_w, W2_b, Wn_w, Wn_b, We_w, We_b,
           bn_gamma, bn_beta, bn_mean, bn_var):
    src = A_edge_index[0]
    dst = A_edge_index[1]
    Hv = H_edge_index[0]
    He = H_edge_index[1]

    x_n, vec = _k1(x, W1_w, W1_b, Wn_w, Wn_b, We_w, We_b)

    a_flat = _k2(vec, src, dst)
    a2d = a_flat.reshape(N, N)
    B = _k3(a2d, H)

    pt_flat = _k4(B.reshape(N * M), Hv, He)
    pt2d = pt_flat.reshape(M, N)

    xl, v2 = _k5(pt2d, x_n, We_w[:, D:])
    v = v2[:, 0]

    q_flat = _k6(vec, v, Hv, He)
    q2d = q_flat.reshape(N, M)

    scale = bn_gamma / jnp.sqrt(bn_var + EPS_BN)
    shift = bn_beta - bn_mean * scale
    x_out = _k7(q2d, xl, x, scale, shift)
    return (x_out, xl)


# speedup vs baseline: 1.4514x; 1.4023x over previous
import functools

import jax
import jax.numpy as jnp
from jax import lax
from jax.experimental import pallas as pl
from jax.experimental.pallas import tpu as pltpu
from jax.experimental.pallas import tpu_sc as plsc

N = 4096
M = 1024
D = 256
EA = 65536
EH = 32768
EPS_BN = 1e-5

NC = 2
NS = 16
L = 16

ZB = 16384
EAT = EA // NS
EHT = EH // NS
HALF_ROWS = N // NC
A_WORDS = (N + 1) * N

_SC_MESH = plsc.VectorSubcoreMesh(core_axis_name="c", subcore_axis_name="s")


def _zero_fill(ref, words):
    z = jnp.zeros((L,), jnp.float32)

    def body(i, _):
        ref[pl.ds(i * 4 * L, L)] = z
        ref[pl.ds(i * 4 * L + L, L)] = z
        ref[pl.ds(i * 4 * L + 2 * L, L)] = z
        ref[pl.ds(i * 4 * L + 3 * L, L)] = z
        return 0

    lax.fori_loop(0, words // (4 * L), body, 0)


def _k1_body(x_ref, w1_ref, b1_ref, wv_ref, bias_ref, xn_ref, vec_ref):
    xn = lax.dot_general(x_ref[...], w1_ref[...],
                         (((1,), (1,)), ((), ())),
                         preferred_element_type=jnp.float32)
    xn = xn + b1_ref[...][0:1, :]
    xn_ref[...] = xn
    vec_ref[...] = lax.dot_general(wv_ref[...], xn,
                                   (((1,), (1,)), ((), ())),
                                   preferred_element_type=jnp.float32) + \
        bias_ref[...][:, 0:1]


def _k1(x, W1_w, W1_b, Wn_w, Wn_b, We_w, We_b):
    wv8 = jnp.zeros((8, D), jnp.float32)
    wv8 = wv8.at[0, :].set(Wn_w[0, :D])
    wv8 = wv8.at[1, :].set(Wn_w[0, D:])
    wv8 = wv8.at[2, :].set(We_w[0, :D])
    bias8 = jnp.zeros((8, 1), jnp.float32)
    bias8 = bias8.at[0, 0].set(Wn_b[0])
    bias8 = bias8.at[2, 0].set(We_b[0])
    b1 = W1_b.reshape(1, D)
    blk = 256
    grid = N // blk
    return pl.pallas_call(
        _k1_body,
        grid=(grid,),
        in_specs=[
            pl.BlockSpec((blk, D), lambda i: (i, 0)),
            pl.BlockSpec((D, D), lambda i: (0, 0)),
            pl.BlockSpec((1, D), lambda i: (0, 0)),
            pl.BlockSpec((8, D), lambda i: (0, 0)),
            pl.BlockSpec((8, 1), lambda i: (0, 0)),
        ],
        out_specs=[
            pl.BlockSpec((blk, D), lambda i: (i, 0)),
            pl.BlockSpec((8, blk), lambda i: (0, i)),
        ],
        out_shape=[
            jax.ShapeDtypeStruct((N, D), jnp.float32),
            jax.ShapeDtypeStruct((8, N), jnp.float32),
        ],
    )(x, W1_w, b1, wv8, bias8)


CH = 8
CROWS = 256
CWORDS = CROWS * N
ECHUNK = EA // NS


def _k2_body(vec_hbm, src_hbm, dst_hbm, out_hbm,
             zbuf, a_v, b_v, src_v, dst_v, idx_v, val_v, a_sh, sem):
    c = lax.axis_index("c")
    s = lax.axis_index("s")
    _zero_fill(zbuf, ZB)
    base = s * ECHUNK
    pltpu.sync_copy(src_hbm.at[pl.ds(base, ECHUNK)], src_v)
    pltpu.sync_copy(dst_hbm.at[pl.ds(base, ECHUNK)], dst_v)
    pltpu.sync_copy(vec_hbm.at[0], a_v)
    pltpu.sync_copy(vec_hbm.at[1], b_v)

    def vbody(i, _):
        s16 = src_v[pl.ds(i * L, L)]
        d16 = dst_v[pl.ds(i * L, L)]
        av = plsc.load_gather(a_v, [s16])
        bv = plsc.load_gather(b_v, [d16])
        val_v[pl.ds(i * L, L)] = 1.0 / (1.0 + jnp.exp(-(av + bv)))
        return 0

    lax.fori_loop(0, ECHUNK // L, vbody, 0)
    slab = CWORDS // NS
    trash = CWORDS + s * ECHUNK
    for r in range(CH):
        lo = (c * CH + r) * CROWS
        zc = [
            pltpu.async_copy(zbuf, a_sh.at[pl.ds(s * slab + j * ZB, ZB)], sem)
            for j in range(slab // ZB)
        ]
        for cp in zc:
            cp.wait()
        plsc.subcore_barrier()

        def ibody(i, _):
            s16 = src_v[pl.ds(i * L, L)]
            d16 = dst_v[pl.ds(i * L, L)]
            idx = (s16 - lo) * N + d16
            keep = (s16 >= lo) & (s16 < lo + CROWS)
            idx_v[pl.ds(i * L, L)] = jnp.where(
                keep, idx, trash + i * L + lax.iota(jnp.int32, L))
            return 0

        lax.fori_loop(0, ECHUNK // L, ibody, 0)
        pltpu.sync_copy(val_v, a_sh.at[idx_v])
        plsc.subcore_barrier()
        pltpu.sync_copy(a_sh.at[pl.ds(s * slab, slab)],
                        out_hbm.at[pl.ds(lo * N + s * slab, slab)])


def _k2(vec, src, dst):
    f = pl.kernel(
        _k2_body,
        out_type=jax.ShapeDtypeStruct((N * N,), jnp.float32),
        mesh=_SC_MESH,
        compiler_params=pltpu.CompilerParams(needs_layout_passes=False),
        scratch_types=[
            pltpu.VMEM((ZB,), jnp.float32),
            pltpu.VMEM((N,), jnp.float32),
            pltpu.VMEM((N,), jnp.float32),
            pltpu.VMEM((ECHUNK,), jnp.int32),
            pltpu.VMEM((ECHUNK,), jnp.int32),
            pltpu.VMEM((ECHUNK,), jnp.int32),
            pltpu.VMEM((ECHUNK,), jnp.float32),
            pltpu.VMEM_SHARED((CWORDS + NS * ECHUNK,), jnp.float32),
            pltpu.SemaphoreType.DMA,
        ],
    )
    return f(vec, src, dst)


def _k3_body(a_ref, h_ref, out_ref):
    blk = 256
    a2 = a_ref[...].reshape(blk, N).astype(jnp.bfloat16)
    out_ref[...] = jnp.dot(a2, h_ref[...],
                           preferred_element_type=jnp.float32).reshape(blk * M)


def _k3(a_flat, H):
    blk = 256
    return pl.pallas_call(
        _k3_body,
        grid=(N // blk,),
        in_specs=[
            pl.BlockSpec((blk * N,), lambda i: (i,)),
            pl.BlockSpec((N, M), lambda i: (0, 0)),
        ],
        out_specs=pl.BlockSpec((blk * M,), lambda i: (i,)),
        out_shape=jax.ShapeDtypeStruct((N * M,), jnp.float32),
    )(a_flat, H.astype(jnp.bfloat16))


QROWS_P = M // (2 * NC)
QWORDS_P = QROWS_P * N

def _k4_body(b_hbm, hv_hbm, he_hbm, pt_hbm,
             zbuf, hv_v, he_v, widx_v, w_v, e_v, s_loc, p_v, pidx_v,
             s_sh, pt_sh, sem):
    c = lax.axis_index("c")
    s = lax.axis_index("s")
    _zero_fill(zbuf, ZB)

    @pl.when(s == 0)
    def _():
        pltpu.sync_copy(zbuf.at[pl.ds(0, M)], s_sh)

    slab = QWORDS_P // NS
    pt_zero = [
        pltpu.async_copy(zbuf, pt_sh.at[pl.ds(s * slab + j * ZB, ZB)], sem)
        for j in range(slab // ZB)
    ]
    for cp in pt_zero:
        cp.wait()
    base = s * EHT
    pltpu.sync_copy(hv_hbm.at[pl.ds(base, EHT)], hv_v)
    pltpu.sync_copy(he_hbm.at[pl.ds(base, EHT)], he_v)

    def mkidx(i, _):
        hv16 = hv_v[pl.ds(i * L, L)]
        he16 = he_v[pl.ds(i * L, L)]
        widx_v[pl.ds(i * L, L)] = hv16 * M + he16
        return 0

    lax.fori_loop(0, EHT // L, mkidx, 0)
    pltpu.async_copy(b_hbm.at[widx_v], w_v, sem).wait()

    def expb(i, _):
        e_v[pl.ds(i * L, L)] = jnp.exp(w_v[pl.ds(i * L, L)])
        return 0

    lax.fori_loop(0, EHT // L, expb, 0)
    plsc.subcore_barrier()
    pltpu.sync_copy(e_v, s_sh.at[he_v], add=True)
    plsc.subcore_barrier()
    pltpu.sync_copy(s_sh, s_loc)

    def pbody(i, _):
        he16 = he_v[pl.ds(i * L, L)]
        se = plsc.load_gather(s_loc, [he16])
        p_v[pl.ds(i * L, L)] = e_v[pl.ds(i * L, L)] / (se + 1e-12)
        return 0

    lax.fori_loop(0, EHT // L, pbody, 0)

    trash = QWORDS_P + s * EHT

    for r in range(2):
        q_lo = (c * 2 + r) * QROWS_P

        def qidx(i, _):
            hv16 = hv_v[pl.ds(i * L, L)]
            he16 = he_v[pl.ds(i * L, L)]
            idx = (he16 - q_lo) * N + hv16
            keep = (he16 >= q_lo) & (he16 < q_lo + QROWS_P)
            pidx_v[pl.ds(i * L, L)] = jnp.where(
                keep, idx, trash + i * L + lax.iota(jnp.int32, L))
            return 0

        lax.fori_loop(0, EHT // L, qidx, 0)
        pltpu.sync_copy(p_v, pt_sh.at[pidx_v], add=True)
        plsc.subcore_barrier()
        pltpu.sync_copy(pt_sh.at[pl.ds(s * slab, slab)],
                        pt_hbm.at[pl.ds(q_lo * N + s * slab, slab)])
        if r == 0:
            rez = [
                pltpu.async_copy(zbuf,
                                 pt_sh.at[pl.ds(s * slab + j * ZB, ZB)], sem)
                for j in range(slab // ZB)
            ]
            for cp in rez:
                cp.wait()
            plsc.subcore_barrier()


def _k4(b_flat, Hv, He):
    f = pl.kernel(
        _k4_body,
        out_type=jax.ShapeDtypeStruct((M * N,), jnp.float32),
        mesh=_SC_MESH,
        compiler_params=pltpu.CompilerParams(needs_layout_passes=False),
        scratch_types=[
            pltpu.VMEM((ZB,), jnp.float32),
            pltpu.VMEM((EHT,), jnp.int32),
            pltpu.VMEM((EHT,), jnp.int32),
            pltpu.VMEM((EHT,), jnp.int32),
            pltpu.VMEM((EHT,), jnp.float32),
            pltpu.VMEM((EHT,), jnp.float32),
            pltpu.VMEM((M,), jnp.float32),
            pltpu.VMEM((EHT,), jnp.float32),
            pltpu.VMEM((EHT,), jnp.int32),
            pltpu.VMEM_SHARED((M,), jnp.float32),
            pltpu.VMEM_SHARED((QWORDS_P + EH,), jnp.float32),
            pltpu.SemaphoreType.DMA,
        ],
    )
    return f(b_flat, Hv, He)


def _k5_body(pt_ref, xn_ref, we2_ref, xl_ref, v_ref):
    pt = pt_ref[...].reshape(M, N)
    xl = jnp.dot(pt, xn_ref[...], preferred_element_type=jnp.float32)
    xl_ref[...] = xl
    v_ref[...] = jnp.sum(xl * we2_ref[...][0:1, :], axis=1, keepdims=True)


def _k5(pt_flat, x_n, we2row):
    return pl.pallas_call(
        _k5_body,
        grid=(1,),
        in_specs=[
            pl.BlockSpec((M * N,), lambda i: (0,)),
            pl.BlockSpec((N, D), lambda i: (0, 0)),
            pl.BlockSpec((1, D), lambda i: (0, 0)),
        ],
        out_specs=[
            pl.BlockSpec((M, D), lambda i: (0, 0)),
            pl.BlockSpec((M, 1), lambda i: (0, 0)),
        ],
        out_shape=[
            jax.ShapeDtypeStruct((M, D), jnp.float32),
            jax.ShapeDtypeStruct((M, 1), jnp.float32),
        ],
    )(pt_flat, x_n, we2row)


QROWS_Q = N // (2 * NC)
QWORDS_Q = QROWS_Q * M

def _k6_body(vec_hbm, v_hbm, hv_hbm, he_hbm, q_hbm,
             zbuf, u_v, v_v, hv_v, he_v, e2_v, s2_loc, q_v, qidx_v,
             s2_sh, q_sh, sem):
    c = lax.axis_index("c")
    s = lax.axis_index("s")
    _zero_fill(zbuf, ZB)

    @pl.when(s == 0)
    def _():
        pltpu.sync_copy(zbuf.at[pl.ds(0, N)], s2_sh)

    slab = QWORDS_Q // NS
    qz = [
        pltpu.async_copy(zbuf, q_sh.at[pl.ds(s * slab + j * ZB, ZB)], sem)
        for j in range(slab // ZB)
    ]
    for cp in qz:
        cp.wait()
    base = s * EHT
    pltpu.sync_copy(hv_hbm.at[pl.ds(base, EHT)], hv_v)
    pltpu.sync_copy(he_hbm.at[pl.ds(base, EHT)], he_v)
    pltpu.sync_copy(vec_hbm.at[2], u_v)
    pltpu.sync_copy(v_hbm, v_v)

    def ebody(i, _):
        hv16 = hv_v[pl.ds(i * L, L)]
        he16 = he_v[pl.ds(i * L, L)]
        uv = plsc.load_gather(u_v, [hv16]) + plsc.load_gather(v_v, [he16])
        ae = 1.0 / (1.0 + jnp.exp(-uv))
        e2_v[pl.ds(i * L, L)] = jnp.exp(ae)
        return 0

    lax.fori_loop(0, EHT // L, ebody, 0)
    plsc.subcore_barrier()
    pltpu.sync_copy(e2_v, s2_sh.at[hv_v], add=True)
    plsc.subcore_barrier()
    pltpu.sync_copy(s2_sh, s2_loc)

    def qbody(i, _):
        hv16 = hv_v[pl.ds(i * L, L)]
        se = plsc.load_gather(s2_loc, [hv16])
        q_v[pl.ds(i * L, L)] = e2_v[pl.ds(i * L, L)] / (se + 1e-12)
        return 0

    lax.fori_loop(0, EHT // L, qbody, 0)

    trash = QWORDS_Q + s * EHT

    for r in range(2):
        q_lo = (c * 2 + r) * QROWS_Q

        def qidx(i, _):
            hv16 = hv_v[pl.ds(i * L, L)]
            he16 = he_v[pl.ds(i * L, L)]
            idx = (hv16 - q_lo) * M + he16
            keep = (hv16 >= q_lo) & (hv16 < q_lo + QROWS_Q)
            qidx_v[pl.ds(i * L, L)] = jnp.where(
                keep, idx, trash + i * L + lax.iota(jnp.int32, L))
            return 0

        lax.fori_loop(0, EHT // L, qidx, 0)
        pltpu.sync_copy(q_v, q_sh.at[qidx_v], add=True)
        plsc.subcore_barrier()
        pltpu.sync_copy(q_sh.at[pl.ds(s * slab, slab)],
                        q_hbm.at[pl.ds(q_lo * M + s * slab, slab)])
        if r == 0:
            rez = [
                pltpu.async_copy(zbuf,
                                 q_sh.at[pl.ds(s * slab + j * ZB, ZB)], sem)
                for j in range(slab // ZB)
            ]
            for cp in rez:
                cp.wait()
            plsc.subcore_barrier()


def _k6(vec, v, Hv, He):
    f = pl.kernel(
        _k6_body,
        out_type=jax.ShapeDtypeStruct((N * M,), jnp.float32),
        mesh=_SC_MESH,
        compiler_params=pltpu.CompilerParams(needs_layout_passes=False),
        scratch_types=[
            pltpu.VMEM((ZB,), jnp.float32),
            pltpu.VMEM((N,), jnp.float32),
            pltpu.VMEM((M,), jnp.float32),
            pltpu.VMEM((EHT,), jnp.int32),
            pltpu.VMEM((EHT,), jnp.int32),
            pltpu.VMEM((EHT,), jnp.float32),
            pltpu.VMEM((N,), jnp.float32),
            pltpu.VMEM((EHT,), jnp.float32),
            pltpu.VMEM((EHT,), jnp.int32),
            pltpu.VMEM_SHARED((N,), jnp.float32),
            pltpu.VMEM_SHARED((QWORDS_Q + EH,), jnp.float32),
            pltpu.SemaphoreType.DMA,
        ],
    )
    return f(vec, v, Hv, He)


def _k7_body(q_ref, xl_ref, x_ref, sc_ref, out_ref):
    q2 = q_ref[...].reshape(256, M)
    acc = jnp.dot(q2, xl_ref[...], preferred_element_type=jnp.float32)
    t = jnp.maximum(acc + x_ref[...], 0.0)
    out_ref[...] = t * sc_ref[...][0:1, :] + sc_ref[...][1:2, :]


def _k7(q_flat, xl, x, scale, shift):
    blk = 256
    sc = jnp.stack([scale, shift], axis=0)
    return pl.pallas_call(
        _k7_body,
        grid=(N // blk,),
        in_specs=[
            pl.BlockSpec((blk * M,), lambda i: (i,)),
            pl.BlockSpec((M, D), lambda i: (0, 0)),
            pl.BlockSpec((blk, D), lambda i: (i, 0)),
            pl.BlockSpec((2, D), lambda i: (0, 0)),
        ],
        out_specs=pl.BlockSpec((blk, D), lambda i: (i, 0)),
        out_shape=jax.ShapeDtypeStruct((N, D), jnp.float32),
    )(q_flat, xl, x, sc)


def kernel(x, H_edge_index, H_edge_weight, A_edge_index, A_edge_weight, H, A,
           X_L, W1_w, W1_b, W2_w, W2_b, Wn_w, Wn_b, We_w, We_b,
           bn_gamma, bn_beta, bn_mean, bn_var):
    src = A_edge_index[0]
    dst = A_edge_index[1]
    Hv = H_edge_index[0]
    He = H_edge_index[1]

    x_n, vec = _k1(x, W1_w, W1_b, Wn_w, Wn_b, We_w, We_b)

    a_flat = _k2(vec, src, dst)
    b_flat = _k3(a_flat, H)

    pt_flat = _k4(b_flat, Hv, He)

    xl, v2 = _k5(pt_flat, x_n, We_w[:, D:])
    v = v2[:, 0]

    q_flat = _k6(vec, v, Hv, He)

    scale = bn_gamma / jnp.sqrt(bn_var + EPS_BN)
    shift = bn_beta - bn_mean * scale
    x_out = _k7(q_flat, xl, x, scale, shift)
    return (x_out, xl)
